# Initial kernel scaffold; baseline (speedup 1.0000x reference)
import jax
import jax.numpy as jnp
from jax.experimental import pallas as pl


def kernel(edge_index_drop, edge_index, features, preference, W_mlp, b_mlp, W_mlp1, b_mlp1):
    raise NotImplementedError("write your pallas kernel here")


# speedup vs baseline: 11.1012x; 11.1012x over previous
import functools

import jax
import jax.numpy as jnp
from jax import lax
from jax.experimental import pallas as pl
from jax.experimental.pallas import tpu as pltpu
from jax.experimental.pallas import tpu_sc as plsc

NUM_USER = 20000
NUM_ITEM = 30000
N_NODES = NUM_USER + NUM_ITEM
N_EDGES = 800000
DIM_FEAT = 128
DIM_LATENT = 64
QW = 16

EBLK = 128
NBLK = 6400
PAD_E = NBLK * EBLK
BLK_PER_TILE = NBLK // 16
DEG_BLK_PER_TILE = NBLK // 32
ACC_ROWS = 50048
ROWS_PER_TILE = ACC_ROWS // 16
DUMMY_ROW = N_NODES
ROW_BLK = 1000
GRID = N_NODES // ROW_BLK

_mesh = plsc.VectorSubcoreMesh(core_axis_name="c", subcore_axis_name="s")
_sc_params = pltpu.CompilerParams(use_tc_tiling_on_sc=False)


@functools.partial(
    pl.kernel,
    out_type=jax.ShapeDtypeStruct((2, ACC_ROWS, 16), jnp.float32),
    mesh=_mesh,
    scratch_types=[
        pltpu.VMEM((DEG_BLK_PER_TILE, EBLK), jnp.int32),
        pltpu.VMEM((EBLK, 16), jnp.float32),
        pltpu.VMEM((EBLK, 16), jnp.float32),
        pltpu.VMEM_SHARED((ACC_ROWS, 16), jnp.float32),
    ],
    compiler_params=_sc_params,
)
def _deg_kernel(row2d, out_hbm, idx_v, ones_v, zero_v, acc_sh):
    cid = lax.axis_index("c")
    sid = lax.axis_index("s")

    def _fill(i, _):
        ones_v[i] = jnp.full((16,), 1.0, jnp.float32)
        zero_v[i] = jnp.zeros((16,), jnp.float32)
        return 0

    lax.fori_loop(0, EBLK, _fill, 0)

    base_row = sid * ROWS_PER_TILE

    def _z(k, _):
        pltpu.sync_copy(zero_v, acc_sh.at[pl.ds(base_row + k * EBLK, EBLK)])
        return 0

    lax.fori_loop(0, 24, _z, 0)
    pltpu.sync_copy(zero_v.at[pl.ds(0, 56)],
                    acc_sh.at[pl.ds(base_row + 24 * EBLK, 56)])
    plsc.subcore_barrier()

    blk0 = cid * (NBLK // 2) + sid * DEG_BLK_PER_TILE
    pltpu.sync_copy(row2d.at[pl.ds(blk0, DEG_BLK_PER_TILE)], idx_v)

    def _scat(j, _):
        pltpu.sync_copy(ones_v, acc_sh.at[idx_v.at[j]], add=True)
        return 0

    lax.fori_loop(0, DEG_BLK_PER_TILE, _scat, 0)
    plsc.subcore_barrier()

    pltpu.sync_copy(acc_sh.at[pl.ds(base_row, ROWS_PER_TILE)],
                    out_hbm.at[cid, pl.ds(base_row, ROWS_PER_TILE)])


@functools.partial(
    pl.kernel,
    out_type=jax.ShapeDtypeStruct((4, ACC_ROWS, QW), jnp.float32),
    mesh=_mesh,
    scratch_types=[
        pltpu.VMEM((40, EBLK), jnp.int32),
        pltpu.VMEM((40, EBLK), jnp.int32),
        pltpu.VMEM((EBLK, QW), jnp.float32),
        pltpu.VMEM((EBLK, QW), jnp.float32),
        pltpu.VMEM((EBLK, QW), jnp.float32),
        pltpu.VMEM((EBLK, QW), jnp.float32),
        pltpu.VMEM_SHARED((ACC_ROWS, QW), jnp.float32),
        pltpu.SemaphoreType.DMA,
        pltpu.SemaphoreType.DMA,
        pltpu.SemaphoreType.DMA,
        pltpu.SemaphoreType.DMA,
    ],
    compiler_params=_sc_params,
)
def _conv_kernel(col2d, row2d, y0, y1, y2, y3, out_hbm,
                 colv, rowv, g0, g1, g2, g3, acc_sh, s0, s1, s2, s3):
    cid = lax.axis_index("c")
    sid = lax.axis_index("s")
    bufs = (g0, g1, g2, g3)
    sems = (s0, s1, s2, s3)
    ys = (y0, y1, y2, y3)
    base_row = sid * ROWS_PER_TILE

    blk0 = sid * BLK_PER_TILE

    for p in range(2):
        def _fill(i, _):
            g0[i] = jnp.zeros((16,), jnp.float32)
            return 0

        lax.fori_loop(0, EBLK, _fill, 0)

        def _z(k, _):
            pltpu.sync_copy(g0, acc_sh.at[pl.ds(base_row + k * EBLK, EBLK)])
            return 0

        lax.fori_loop(0, 24, _z, 0)
        pltpu.sync_copy(g0.at[pl.ds(0, 56)],
                        acc_sh.at[pl.ds(base_row + 24 * EBLK, 56)])
        plsc.subcore_barrier()

        def _chunk(ci, _):
            pltpu.sync_copy(col2d.at[pl.ds(blk0 + ci * 40, 40)], colv)
            pltpu.sync_copy(row2d.at[pl.ds(blk0 + ci * 40, 40)], rowv)

            def _group(g, _):
                descs = []
                for i in range(4):
                    jc = g * 4 + i

                    @pl.when(cid == 0)
                    def _(jc=jc, buf=bufs[i], sem=sems[i]):
                        pltpu.async_copy(ys[p].at[colv.at[jc]], buf, sem)

                    @pl.when(cid == 1)
                    def _(jc=jc, buf=bufs[i], sem=sems[i]):
                        pltpu.async_copy(ys[2 + p].at[colv.at[jc]], buf, sem)

                    descs.append(
                        pltpu.make_async_copy(ys[p].at[colv.at[jc]],
                                              bufs[i], sems[i]))
                for i in range(4):
                    jc = g * 4 + i
                    descs[i].wait()
                    pltpu.sync_copy(bufs[i], acc_sh.at[rowv.at[jc]], add=True)
                return 0

            lax.fori_loop(0, 10, _group, 0)
            return 0

        lax.fori_loop(0, 10, _chunk, 0)
        plsc.subcore_barrier()

        pltpu.sync_copy(acc_sh.at[pl.ds(base_row, ROWS_PER_TILE)],
                        out_hbm.at[2 * cid + p, pl.ds(base_row, ROWS_PER_TILE)])
        if p == 0:
            pass


def _dis_from(degp):
    deg = degp[0, :, 0:1] + degp[1, :, 0:1]
    return jnp.where(deg > 0, lax.rsqrt(jnp.maximum(deg, 1e-30)), 0.0)


def _mlp_body(feat_ref, pref_ref, degp_ref, w_ref, b_ref, w1_ref, b1_ref,
              x_ref, y0_ref, y1_ref, y2_ref, y3_ref):
    i = pl.program_id(0)
    yrefs = (y0_ref, y1_ref, y2_ref, y3_ref)

    def _finish(xb):
        nrm = jnp.maximum(
            jnp.sqrt(jnp.sum(xb * xb, axis=1, keepdims=True)), 1e-12)
        xb = xb / nrm
        dis = _dis_from(degp_ref[...])
        y = xb * dis
        x_ref[...] = xb
        for q in range(4):
            yrefs[q][...] = y[:, q * QW:(q + 1) * QW]

    @pl.when(i < NUM_USER // ROW_BLK)
    def _():
        _finish(pref_ref[...])

    @pl.when(i >= NUM_USER // ROW_BLK)
    def _():
        h = feat_ref[...] @ w_ref[...] + b_ref[0]
        h = jnp.where(h > 0, h, 0.01 * h)
        _finish(h @ w1_ref[...] + b1_ref[0])


def _scale_body(s_ref, degp_ref, h_ref, y0_ref, y1_ref, y2_ref, y3_ref):
    s = jnp.concatenate([s_ref[0], s_ref[1], s_ref[2], s_ref[3]], axis=1)
    dis = _dis_from(degp_ref[...])
    h = s * dis
    y2 = h * dis
    h_ref[...] = h
    for q, yr in enumerate((y0_ref, y1_ref, y2_ref, y3_ref)):
        yr[...] = y2[:, q * QW:(q + 1) * QW]


def _final_body(h_ref, x_ref, s_ref, degp_ref, out_ref):
    s = jnp.concatenate([s_ref[0], s_ref[1], s_ref[2], s_ref[3]], axis=1)
    dis = _dis_from(degp_ref[...])
    out_ref[...] = h_ref[...] + x_ref[...] + s * dis


def _degp_spec():
    return pl.BlockSpec((2, ROW_BLK, 16), lambda i: (0, i, 0))


def _s_spec():
    return pl.BlockSpec((4, ROW_BLK, QW), lambda i: (0, i, 0))


def _row_spec(width):
    return pl.BlockSpec((ROW_BLK, width), lambda i: (i, 0))


def _quarter_outs():
    return [jax.ShapeDtypeStruct((N_NODES, QW), jnp.float32)
            for _ in range(4)]


def kernel(edge_index_drop, edge_index, features, preference,
           W_mlp, b_mlp, W_mlp1, b_mlp1):
    del edge_index_drop
    row = edge_index[0].astype(jnp.int32)
    col = edge_index[1].astype(jnp.int32)
    pad = PAD_E - N_EDGES
    row2d = jnp.concatenate(
        [row, jnp.full((pad,), DUMMY_ROW, jnp.int32)]).reshape(NBLK, EBLK)
    col2d = jnp.concatenate(
        [col, jnp.zeros((pad,), jnp.int32)]).reshape(NBLK, EBLK)

    degp = _deg_kernel(row2d)

    x, ya0, ya1, ya2, ya3 = pl.pallas_call(
        _mlp_body,
        grid=(GRID,),
        in_specs=[
            pl.BlockSpec((ROW_BLK, DIM_FEAT),
                         lambda i: (jnp.clip(i - NUM_USER // ROW_BLK, 0,
                                             NUM_ITEM // ROW_BLK - 1), 0)),
            pl.BlockSpec((ROW_BLK, DIM_LATENT),
                         lambda i: (jnp.minimum(i, NUM_USER // ROW_BLK - 1), 0)),
            _degp_spec(),
            pl.BlockSpec((DIM_FEAT, 4 * DIM_LATENT), lambda i: (0, 0)),
            pl.BlockSpec((1, 4 * DIM_LATENT), lambda i: (0, 0)),
            pl.BlockSpec((4 * DIM_LATENT, DIM_LATENT), lambda i: (0, 0)),
            pl.BlockSpec((1, DIM_LATENT), lambda i: (0, 0)),
        ],
        out_specs=[_row_spec(DIM_LATENT)] + [_row_spec(QW)] * 4,
        out_shape=[jax.ShapeDtypeStruct((N_NODES, DIM_LATENT), jnp.float32)]
        + _quarter_outs(),
    )(features, preference, degp, W_mlp, b_mlp.reshape(1, -1),
      W_mlp1, b_mlp1.reshape(1, -1))

    s1 = _conv_kernel(col2d, row2d, ya0, ya1, ya2, ya3)

    h, yb0, yb1, yb2, yb3 = pl.pallas_call(
        _scale_body,
        grid=(GRID,),
        in_specs=[_s_spec(), _degp_spec()],
        out_specs=[_row_spec(DIM_LATENT)] + [_row_spec(QW)] * 4,
        out_shape=[jax.ShapeDtypeStruct((N_NODES, DIM_LATENT), jnp.float32)]
        + _quarter_outs(),
    )(s1, degp)

    s2 = _conv_kernel(col2d, row2d, yb0, yb1, yb2, yb3)

    x_hat = pl.pallas_call(
        _final_body,
        grid=(GRID,),
        in_specs=[_row_spec(DIM_LATENT), _row_spec(DIM_LATENT),
                  _s_spec(), _degp_spec()],
        out_specs=_row_spec(DIM_LATENT),
        out_shape=jax.ShapeDtypeStruct((N_NODES, DIM_LATENT), jnp.float32),
    )(h, x, s2, degp)

    return (x_hat, preference)


# speedup vs baseline: 12.7481x; 1.1484x over previous
import functools

import jax
import jax.numpy as jnp
from jax import lax
from jax.experimental import pallas as pl
from jax.experimental.pallas import tpu as pltpu
from jax.experimental.pallas import tpu_sc as plsc

NUM_USER = 20000
NUM_ITEM = 30000
N_NODES = NUM_USER + NUM_ITEM
N_EDGES = 800000
DIM_FEAT = 128
DIM_LATENT = 64
QW = 16

EBLK = 128
NBLK = 6400
PAD_E = NBLK * EBLK
BLK_PER_TILE = NBLK // 16
DEG_BLK_PER_TILE = NBLK // 32
ACC_ROWS = 51200
ROWS_PER_TILE = ACC_ROWS // 16
DUMMY_ROW = N_NODES
FQ = ACC_ROWS * QW // 128
ROW_BLK = 1000
GRID = N_NODES // ROW_BLK
PBLK = 1600
PGRID = ACC_ROWS // PBLK
WBLK = PBLK * QW // 128

_mesh = plsc.VectorSubcoreMesh(core_axis_name="c", subcore_axis_name="s")
_sc_params = pltpu.CompilerParams(use_tc_tiling_on_sc=False)


@functools.partial(
    pl.kernel,
    out_type=jax.ShapeDtypeStruct((2, ACC_ROWS, 16), jnp.float32),
    mesh=_mesh,
    scratch_types=[
        pltpu.VMEM((40, EBLK), jnp.int32),
        pltpu.VMEM((EBLK, 16), jnp.float32),
        pltpu.VMEM((EBLK, 16), jnp.float32),
        pltpu.VMEM_SHARED((ACC_ROWS, 16), jnp.float32),
    ],
    compiler_params=_sc_params,
)
def _deg_kernel(row2d, out_hbm, idx_v, ones_v, zero_v, acc_sh):
    cid = lax.axis_index("c")
    sid = lax.axis_index("s")

    def _fill(i, _):
        ones_v[i] = jnp.full((16,), 1.0, jnp.float32)
        zero_v[i] = jnp.zeros((16,), jnp.float32)
        return 0

    lax.fori_loop(0, EBLK, _fill, 0)

    base_row = sid * ROWS_PER_TILE

    def _z(k, _):
        pltpu.sync_copy(zero_v, acc_sh.at[pl.ds(base_row + k * EBLK, EBLK)])
        return 0

    lax.fori_loop(0, 25, _z, 0)
    plsc.subcore_barrier()

    blk0 = cid * (NBLK // 2) + sid * DEG_BLK_PER_TILE

    def _chunk(ci, _):
        pltpu.sync_copy(row2d.at[pl.ds(blk0 + ci * 40, 40)], idx_v)

        def _scat(j, _):
            pltpu.sync_copy(ones_v, acc_sh.at[idx_v.at[j]], add=True)
            return 0

        lax.fori_loop(0, 40, _scat, 0)
        return 0

    lax.fori_loop(0, 5, _chunk, 0)
    plsc.subcore_barrier()

    pltpu.sync_copy(acc_sh.at[pl.ds(base_row, ROWS_PER_TILE)],
                    out_hbm.at[cid, pl.ds(base_row, ROWS_PER_TILE)])


@functools.partial(
    pl.kernel,
    out_type=jax.ShapeDtypeStruct((4, ACC_ROWS, QW), jnp.float32),
    mesh=_mesh,
    scratch_types=[
        pltpu.VMEM((40, EBLK), jnp.int32),
        pltpu.VMEM((40, EBLK), jnp.int32),
        pltpu.VMEM((EBLK, QW), jnp.float32),
        pltpu.VMEM((EBLK, QW), jnp.float32),
        pltpu.VMEM((EBLK, QW), jnp.float32),
        pltpu.VMEM((EBLK, QW), jnp.float32),
        pltpu.VMEM_SHARED((ACC_ROWS, QW), jnp.float32),
        pltpu.SemaphoreType.DMA,
        pltpu.SemaphoreType.DMA,
        pltpu.SemaphoreType.DMA,
        pltpu.SemaphoreType.DMA,
    ],
    compiler_params=_sc_params,
)
def _conv_kernel(col2d, row2d, y0, y1, y2, y3, out_hbm,
                 colv, rowv, g0, g1, g2, g3, acc_sh, s0, s1, s2, s3):
    cid = lax.axis_index("c")
    sid = lax.axis_index("s")
    bufs = (g0, g1, g2, g3)
    sems = (s0, s1, s2, s3)
    ys = (y0, y1, y2, y3)
    base_row = sid * ROWS_PER_TILE

    blk0 = sid * BLK_PER_TILE

    for p in range(2):
        def _fill(i, _):
            g0[i] = jnp.zeros((16,), jnp.float32)
            return 0

        lax.fori_loop(0, EBLK, _fill, 0)

        def _z(k, _):
            pltpu.sync_copy(g0, acc_sh.at[pl.ds(base_row + k * EBLK, EBLK)])
            return 0

        lax.fori_loop(0, 25, _z, 0)
        plsc.subcore_barrier()

        def _chunk(ci, _):
            pltpu.sync_copy(col2d.at[pl.ds(blk0 + ci * 40, 40)], colv)
            pltpu.sync_copy(row2d.at[pl.ds(blk0 + ci * 40, 40)], rowv)

            def _group(g, _):
                descs = []
                for i in range(4):
                    jc = g * 4 + i

                    @pl.when(cid == 0)
                    def _(jc=jc, buf=bufs[i], sem=sems[i]):
                        pltpu.async_copy(ys[p].at[colv.at[jc]], buf, sem)

                    @pl.when(cid == 1)
                    def _(jc=jc, buf=bufs[i], sem=sems[i]):
                        pltpu.async_copy(ys[2 + p].at[colv.at[jc]], buf, sem)

                    descs.append(
                        pltpu.make_async_copy(ys[p].at[colv.at[jc]],
                                              bufs[i], sems[i]))
                for i in range(4):
                    jc = g * 4 + i
                    descs[i].wait()
                    pltpu.sync_copy(bufs[i], acc_sh.at[rowv.at[jc]], add=True)
                return 0

            lax.fori_loop(0, 10, _group, 0)
            return 0

        lax.fori_loop(0, 10, _chunk, 0)
        plsc.subcore_barrier()

        pltpu.sync_copy(acc_sh.at[pl.ds(base_row, ROWS_PER_TILE)],
                        out_hbm.at[2 * cid + p, pl.ds(base_row, ROWS_PER_TILE)])


def _n2w(x):
    x3 = jnp.reshape(x, (WBLK, 8, QW))
    return jnp.concatenate([x3[:, s, :] for s in range(8)], axis=1)


def _w2n(w):
    st = jnp.stack([w[:, s * QW:(s + 1) * QW] for s in range(8)], axis=1)
    return jnp.reshape(st, (PBLK, QW))


def _dis_wide(dp0, dp1):
    deg = dp0 + dp1
    return jnp.where(deg > 0, lax.rsqrt(jnp.maximum(deg, 1e-30)), 0.0)


def _mlp_body(feat_ref, pref_ref, w_ref, b_ref, w1_ref, b1_ref, x_ref):
    i = pl.program_id(0)

    def _finish(xb):
        nrm = jnp.maximum(
            jnp.sqrt(jnp.sum(xb * xb, axis=1, keepdims=True)), 1e-12)
        x_ref[...] = xb / nrm

    @pl.when(i < NUM_USER // ROW_BLK)
    def _():
        _finish(pref_ref[...])

    @pl.when(i >= NUM_USER // ROW_BLK)
    def _():
        h = feat_ref[...] @ w_ref[...] + b_ref[0]
        h = jnp.where(h > 0, h, 0.01 * h)
        _finish(h @ w1_ref[...] + b1_ref[0])


def _pack_body(x_ref, dp0_ref, dp1_ref, y0_ref, y1_ref, y2_ref, y3_ref):
    dis = _dis_wide(dp0_ref[...], dp1_ref[...])
    x = x_ref[...]
    for q, yr in enumerate((y0_ref, y1_ref, y2_ref, y3_ref)):
        yr[...] = _n2w(x[:, q * QW:(q + 1) * QW]) * dis


def _scale_body(s0_ref, s1_ref, s2_ref, s3_ref, dp0_ref, dp1_ref,
                h0_ref, h1_ref, h2_ref, h3_ref,
                y0_ref, y1_ref, y2_ref, y3_ref):
    dis = _dis_wide(dp0_ref[...], dp1_ref[...])
    for sr, hr, yr in zip((s0_ref, s1_ref, s2_ref, s3_ref),
                          (h0_ref, h1_ref, h2_ref, h3_ref),
                          (y0_ref, y1_ref, y2_ref, y3_ref)):
        h = sr[...] * dis
        hr[...] = h
        yr[...] = h * dis


def _final_body(x_ref, h0_ref, h1_ref, h2_ref, h3_ref,
                s0_ref, s1_ref, s2_ref, s3_ref, dp0_ref, dp1_ref, out_ref):
    dis = _dis_wide(dp0_ref[...], dp1_ref[...])
    cols = []
    for hr, sr in zip((h0_ref, h1_ref, h2_ref, h3_ref),
                      (s0_ref, s1_ref, s2_ref, s3_ref)):
        cols.append(_w2n(hr[...] + sr[...] * dis))
    out_ref[...] = x_ref[...] + jnp.concatenate(cols, axis=1)


def _wide_spec(plane_of_grid):
    n = FQ // WBLK
    return pl.BlockSpec((WBLK, 128), lambda i, p=plane_of_grid: (p * n + i, 0))


def _q_spec():
    return pl.BlockSpec((WBLK, 128), lambda i: (i, 0))


def _x_spec():
    return pl.BlockSpec((PBLK, DIM_LATENT), lambda i: (i, 0))


def _q_outs():
    return [jax.ShapeDtypeStruct((FQ, 128), jnp.float32) for _ in range(4)]


def kernel(edge_index_drop, edge_index, features, preference,
           W_mlp, b_mlp, W_mlp1, b_mlp1):
    del edge_index_drop
    row = edge_index[0].astype(jnp.int32)
    col = edge_index[1].astype(jnp.int32)
    pad = PAD_E - N_EDGES
    row2d = jnp.concatenate(
        [row, jnp.full((pad,), DUMMY_ROW, jnp.int32)]).reshape(NBLK, EBLK)
    col2d = jnp.concatenate(
        [col, jnp.zeros((pad,), jnp.int32)]).reshape(NBLK, EBLK)

    degp = _deg_kernel(row2d)
    degp_flat = jnp.reshape(degp, (2 * FQ, 128))

    x = pl.pallas_call(
        _mlp_body,
        grid=(GRID,),
        in_specs=[
            pl.BlockSpec((ROW_BLK, DIM_FEAT),
                         lambda i: (jnp.clip(i - NUM_USER // ROW_BLK, 0,
                                             NUM_ITEM // ROW_BLK - 1), 0)),
            pl.BlockSpec((ROW_BLK, DIM_LATENT),
                         lambda i: (jnp.minimum(i, NUM_USER // ROW_BLK - 1), 0)),
            pl.BlockSpec((DIM_FEAT, 4 * DIM_LATENT), lambda i: (0, 0)),
            pl.BlockSpec((1, 4 * DIM_LATENT), lambda i: (0, 0)),
            pl.BlockSpec((4 * DIM_LATENT, DIM_LATENT), lambda i: (0, 0)),
            pl.BlockSpec((1, DIM_LATENT), lambda i: (0, 0)),
        ],
        out_specs=pl.BlockSpec((ROW_BLK, DIM_LATENT), lambda i: (i, 0)),
        out_shape=jax.ShapeDtypeStruct((N_NODES, DIM_LATENT), jnp.float32),
    )(features, preference, W_mlp, b_mlp.reshape(1, -1),
      W_mlp1, b_mlp1.reshape(1, -1))

    ya = pl.pallas_call(
        _pack_body,
        grid=(PGRID,),
        in_specs=[_x_spec(), _wide_spec(0), _wide_spec(1)],
        out_specs=[_q_spec()] * 4,
        out_shape=_q_outs(),
    )(x, degp_flat, degp_flat)

    s1 = _conv_kernel(col2d, row2d,
                      *(jnp.reshape(y, (ACC_ROWS, QW)) for y in ya))
    s1_flat = jnp.reshape(s1, (4 * FQ, 128))

    h_and_y2 = pl.pallas_call(
        _scale_body,
        grid=(PGRID,),
        in_specs=[_wide_spec(0), _wide_spec(1), _wide_spec(2), _wide_spec(3),
                  _wide_spec(0), _wide_spec(1)],
        out_specs=[_q_spec()] * 8,
        out_shape=_q_outs() + _q_outs(),
    )(s1_flat, s1_flat, s1_flat, s1_flat, degp_flat, degp_flat)
    hq, yb = h_and_y2[:4], h_and_y2[4:]

    s2 = _conv_kernel(col2d, row2d,
                      *(jnp.reshape(y, (ACC_ROWS, QW)) for y in yb))
    s2_flat = jnp.reshape(s2, (4 * FQ, 128))

    x_hat = pl.pallas_call(
        _final_body,
        grid=(PGRID,),
        in_specs=[_x_spec()] + [_q_spec()] * 4
        + [_wide_spec(0), _wide_spec(1), _wide_spec(2), _wide_spec(3),
           _wide_spec(0), _wide_spec(1)],
        out_specs=_x_spec(),
        out_shape=jax.ShapeDtypeStruct((N_NODES, DIM_LATENT), jnp.float32),
    )(x, *hq, s2_flat, s2_flat, s2_flat, s2_flat, degp_flat, degp_flat)

    return (x_hat, preference)


# speedup vs baseline: 14.1995x; 1.1138x over previous
import functools

import jax
import jax.numpy as jnp
from jax import lax
from jax.experimental import pallas as pl
from jax.experimental.pallas import tpu as pltpu
from jax.experimental.pallas import tpu_sc as plsc

NUM_USER = 20000
NUM_ITEM = 30000
N_NODES = NUM_USER + NUM_ITEM
N_EDGES = 800000
DIM_FEAT = 128
DIM_LATENT = 64
QW = 16

EBLK = 128
NBLK = 6400
PAD_E = NBLK * EBLK
BLK_PER_TILE = NBLK // 16
DEG_BLK_PER_TILE = NBLK // 32
ACC_ROWS = 51200
ROWS_PER_TILE = ACC_ROWS // 16
DUMMY_ROW = N_NODES
FQ = ACC_ROWS * QW // 128
ROW_BLK = 1000
GRID = N_NODES // ROW_BLK
PBLK = 1600
PGRID = ACC_ROWS // PBLK
WBLK = PBLK * QW // 128

_mesh = plsc.VectorSubcoreMesh(core_axis_name="c", subcore_axis_name="s")
_sc_params = pltpu.CompilerParams(use_tc_tiling_on_sc=False)


@functools.partial(
    pl.kernel,
    out_type=jax.ShapeDtypeStruct((2, ACC_ROWS, 16), jnp.float32),
    mesh=_mesh,
    scratch_types=[
        pltpu.VMEM((40, EBLK), jnp.int32),
        pltpu.VMEM((EBLK, 16), jnp.float32),
        pltpu.VMEM((EBLK, 16), jnp.float32),
        pltpu.VMEM_SHARED((ACC_ROWS, 16), jnp.float32),
    ],
    compiler_params=_sc_params,
)
def _deg_kernel(row2d, out_hbm, idx_v, ones_v, zero_v, acc_sh):
    cid = lax.axis_index("c")
    sid = lax.axis_index("s")

    def _fill(i, _):
        ones_v[i] = jnp.full((16,), 1.0, jnp.float32)
        zero_v[i] = jnp.zeros((16,), jnp.float32)
        return 0

    lax.fori_loop(0, EBLK, _fill, 0)

    base_row = sid * ROWS_PER_TILE

    def _z(k, _):
        pltpu.sync_copy(zero_v, acc_sh.at[pl.ds(base_row + k * EBLK, EBLK)])
        return 0

    lax.fori_loop(0, 25, _z, 0)
    plsc.subcore_barrier()

    blk0 = cid * (NBLK // 2) + sid * DEG_BLK_PER_TILE

    def _chunk(ci, _):
        pltpu.sync_copy(row2d.at[pl.ds(blk0 + ci * 40, 40)], idx_v)

        def _scat(j, _):
            pltpu.sync_copy(ones_v, acc_sh.at[idx_v.at[j]], add=True)
            return 0

        lax.fori_loop(0, 40, _scat, 0)
        return 0

    lax.fori_loop(0, 5, _chunk, 0)
    plsc.subcore_barrier()

    pltpu.sync_copy(acc_sh.at[pl.ds(base_row, ROWS_PER_TILE)],
                    out_hbm.at[cid, pl.ds(base_row, ROWS_PER_TILE)])


@functools.partial(
    pl.kernel,
    out_type=jax.ShapeDtypeStruct((4, ACC_ROWS, QW), jnp.float32),
    mesh=_mesh,
    scratch_types=(
        [pltpu.VMEM((40, EBLK), jnp.int32)] * 2
        + [pltpu.VMEM((EBLK, QW), jnp.float32)] * 8
        + [pltpu.VMEM_SHARED((ACC_ROWS, QW), jnp.float32)]
        + [pltpu.SemaphoreType.DMA] * 16
    ),
    compiler_params=_sc_params,
)
def _conv_kernel(col2d, row2d, y0, y1, y2, y3, out_hbm,
                 colv, rowv, *rest):
    bufs = rest[:8]
    acc_sh = rest[8]
    gsems = rest[9:17]
    tsems = rest[17:25]
    cid = lax.axis_index("c")
    sid = lax.axis_index("s")
    ys = (y0, y1, y2, y3)
    base_row = sid * ROWS_PER_TILE

    def _wait_scat(i):
        pltpu.make_async_copy(bufs[i], acc_sh.at[rowv.at[0]], tsems[i]).wait()

    blk0 = sid * BLK_PER_TILE

    for p in range(2):
        def _fill(i, _):
            bufs[0][i] = jnp.zeros((16,), jnp.float32)
            return 0

        lax.fori_loop(0, EBLK, _fill, 0)

        def _z(k, _):
            pltpu.sync_copy(bufs[0],
                            acc_sh.at[pl.ds(base_row + k * EBLK, EBLK)])
            return 0

        lax.fori_loop(0, 25, _z, 0)
        plsc.subcore_barrier()

        def _chunk(ci, _):
            @pl.when(ci > 0)
            def _():
                for i in range(8):
                    _wait_scat(i)

            pltpu.sync_copy(col2d.at[pl.ds(blk0 + ci * 40, 40)], colv)
            pltpu.sync_copy(row2d.at[pl.ds(blk0 + ci * 40, 40)], rowv)

            def _group(g, _):
                for par in range(2):
                    @pl.when(g % 2 == par)
                    def _(par=par):
                        b0 = par * 4
                        for i in range(4):
                            jc = g * 4 + i

                            @pl.when(g >= 2)
                            def _(i=i, b0=b0):
                                _wait_scat(b0 + i)

                            @pl.when(cid == 0)
                            def _(jc=jc, k=b0 + i):
                                pltpu.async_copy(ys[p].at[colv.at[jc]],
                                                 bufs[k], gsems[k])

                            @pl.when(cid == 1)
                            def _(jc=jc, k=b0 + i):
                                pltpu.async_copy(ys[2 + p].at[colv.at[jc]],
                                                 bufs[k], gsems[k])
                        for i in range(4):
                            jc = g * 4 + i
                            k = b0 + i
                            pltpu.make_async_copy(
                                ys[p].at[colv.at[jc]], bufs[k],
                                gsems[k]).wait()
                            pltpu.async_copy(
                                bufs[k], acc_sh.at[rowv.at[jc]],
                                tsems[k], add=True)
                return 0

            lax.fori_loop(0, 10, _group, 0)
            return 0

        lax.fori_loop(0, 10, _chunk, 0)
        for i in range(8):
            _wait_scat(i)
        plsc.subcore_barrier()

        pltpu.sync_copy(acc_sh.at[pl.ds(base_row, ROWS_PER_TILE)],
                        out_hbm.at[2 * cid + p, pl.ds(base_row, ROWS_PER_TILE)])


def _n2w(x):
    x3 = jnp.reshape(x, (WBLK, 8, QW))
    return jnp.concatenate([x3[:, s, :] for s in range(8)], axis=1)


def _w2n(w):
    st = jnp.stack([w[:, s * QW:(s + 1) * QW] for s in range(8)], axis=1)
    return jnp.reshape(st, (PBLK, QW))


def _dis_wide(dp0, dp1):
    deg = dp0 + dp1
    return jnp.where(deg > 0, lax.rsqrt(jnp.maximum(deg, 1e-30)), 0.0)


def _mlp_body(feat_ref, pref_ref, w_ref, b_ref, w1_ref, b1_ref, x_ref):
    i = pl.program_id(0)

    def _finish(xb):
        nrm = jnp.maximum(
            jnp.sqrt(jnp.sum(xb * xb, axis=1, keepdims=True)), 1e-12)
        x_ref[...] = xb / nrm

    @pl.when(i < NUM_USER // ROW_BLK)
    def _():
        _finish(pref_ref[...])

    @pl.when(i >= NUM_USER // ROW_BLK)
    def _():
        h = feat_ref[...] @ w_ref[...] + b_ref[0]
        h = jnp.where(h > 0, h, 0.01 * h)
        _finish(h @ w1_ref[...] + b1_ref[0])


def _pack_body(x_ref, dp0_ref, dp1_ref, y0_ref, y1_ref, y2_ref, y3_ref):
    dis = _dis_wide(dp0_ref[...], dp1_ref[...])
    x = x_ref[...]
    for q, yr in enumerate((y0_ref, y1_ref, y2_ref, y3_ref)):
        yr[...] = _n2w(x[:, q * QW:(q + 1) * QW]) * dis


def _scale_body(s0_ref, s1_ref, s2_ref, s3_ref, dp0_ref, dp1_ref,
                h0_ref, h1_ref, h2_ref, h3_ref,
                y0_ref, y1_ref, y2_ref, y3_ref):
    dis = _dis_wide(dp0_ref[...], dp1_ref[...])
    for sr, hr, yr in zip((s0_ref, s1_ref, s2_ref, s3_ref),
                          (h0_ref, h1_ref, h2_ref, h3_ref),
                          (y0_ref, y1_ref, y2_ref, y3_ref)):
        h = sr[...] * dis
        hr[...] = h
        yr[...] = h * dis


def _final_body(x_ref, h0_ref, h1_ref, h2_ref, h3_ref,
                s0_ref, s1_ref, s2_ref, s3_ref, dp0_ref, dp1_ref, out_ref):
    dis = _dis_wide(dp0_ref[...], dp1_ref[...])
    cols = []
    for hr, sr in zip((h0_ref, h1_ref, h2_ref, h3_ref),
                      (s0_ref, s1_ref, s2_ref, s3_ref)):
        cols.append(_w2n(hr[...] + sr[...] * dis))
    out_ref[...] = x_ref[...] + jnp.concatenate(cols, axis=1)


def _wide_spec(plane_of_grid):
    n = FQ // WBLK
    return pl.BlockSpec((WBLK, 128), lambda i, p=plane_of_grid: (p * n + i, 0))


def _q_spec():
    return pl.BlockSpec((WBLK, 128), lambda i: (i, 0))


def _x_spec():
    return pl.BlockSpec((PBLK, DIM_LATENT), lambda i: (i, 0))


def _q_outs():
    return [jax.ShapeDtypeStruct((FQ, 128), jnp.float32) for _ in range(4)]


def kernel(edge_index_drop, edge_index, features, preference,
           W_mlp, b_mlp, W_mlp1, b_mlp1):
    del edge_index_drop
    row = edge_index[0].astype(jnp.int32)
    col = edge_index[1].astype(jnp.int32)
    pad = PAD_E - N_EDGES
    row2d = jnp.concatenate(
        [row, jnp.full((pad,), DUMMY_ROW, jnp.int32)]).reshape(NBLK, EBLK)
    col2d = jnp.concatenate(
        [col, jnp.zeros((pad,), jnp.int32)]).reshape(NBLK, EBLK)

    degp = _deg_kernel(row2d)
    degp_flat = jnp.reshape(degp, (2 * FQ, 128))

    x = pl.pallas_call(
        _mlp_body,
        grid=(GRID,),
        in_specs=[
            pl.BlockSpec((ROW_BLK, DIM_FEAT),
                         lambda i: (jnp.clip(i - NUM_USER // ROW_BLK, 0,
                                             NUM_ITEM // ROW_BLK - 1), 0)),
            pl.BlockSpec((ROW_BLK, DIM_LATENT),
                         lambda i: (jnp.minimum(i, NUM_USER // ROW_BLK - 1), 0)),
            pl.BlockSpec((DIM_FEAT, 4 * DIM_LATENT), lambda i: (0, 0)),
            pl.BlockSpec((1, 4 * DIM_LATENT), lambda i: (0, 0)),
            pl.BlockSpec((4 * DIM_LATENT, DIM_LATENT), lambda i: (0, 0)),
            pl.BlockSpec((1, DIM_LATENT), lambda i: (0, 0)),
        ],
        out_specs=pl.BlockSpec((ROW_BLK, DIM_LATENT), lambda i: (i, 0)),
        out_shape=jax.ShapeDtypeStruct((N_NODES, DIM_LATENT), jnp.float32),
    )(features, preference, W_mlp, b_mlp.reshape(1, -1),
      W_mlp1, b_mlp1.reshape(1, -1))

    ya = pl.pallas_call(
        _pack_body,
        grid=(PGRID,),
        in_specs=[_x_spec(), _wide_spec(0), _wide_spec(1)],
        out_specs=[_q_spec()] * 4,
        out_shape=_q_outs(),
    )(x, degp_flat, degp_flat)

    s1 = _conv_kernel(col2d, row2d,
                      *(jnp.reshape(y, (ACC_ROWS, QW)) for y in ya))
    s1_flat = jnp.reshape(s1, (4 * FQ, 128))

    h_and_y2 = pl.pallas_call(
        _scale_body,
        grid=(PGRID,),
        in_specs=[_wide_spec(0), _wide_spec(1), _wide_spec(2), _wide_spec(3),
                  _wide_spec(0), _wide_spec(1)],
        out_specs=[_q_spec()] * 8,
        out_shape=_q_outs() + _q_outs(),
    )(s1_flat, s1_flat, s1_flat, s1_flat, degp_flat, degp_flat)
    hq, yb = h_and_y2[:4], h_and_y2[4:]

    s2 = _conv_kernel(col2d, row2d,
                      *(jnp.reshape(y, (ACC_ROWS, QW)) for y in yb))
    s2_flat = jnp.reshape(s2, (4 * FQ, 128))

    x_hat = pl.pallas_call(
        _final_body,
        grid=(PGRID,),
        in_specs=[_x_spec()] + [_q_spec()] * 4
        + [_wide_spec(0), _wide_spec(1), _wide_spec(2), _wide_spec(3),
           _wide_spec(0), _wide_spec(1)],
        out_specs=_x_spec(),
        out_shape=jax.ShapeDtypeStruct((N_NODES, DIM_LATENT), jnp.float32),
    )(x, *hq, s2_flat, s2_flat, s2_flat, s2_flat, degp_flat, degp_flat)

    return (x_hat, preference)


# speedup vs baseline: 14.4602x; 1.0184x over previous
import functools

import jax
import jax.numpy as jnp
from jax import lax
from jax.experimental import pallas as pl
from jax.experimental.pallas import tpu as pltpu
from jax.experimental.pallas import tpu_sc as plsc

NUM_USER = 20000
NUM_ITEM = 30000
N_NODES = NUM_USER + NUM_ITEM
N_EDGES = 800000
DIM_FEAT = 128
DIM_LATENT = 64
QW = 16

EBLK = 128
NBLK = 6400
PAD_E = NBLK * EBLK
BLK_PER_TILE = NBLK // 16
DEG_BLK_PER_TILE = NBLK // 32
ACC_ROWS = 51200
ROWS_PER_TILE = ACC_ROWS // 16
DUMMY_ROW = N_NODES
FQ = ACC_ROWS * QW // 128
ROW_BLK = 2000
GRID = N_NODES // ROW_BLK
PBLK = 1600
PGRID = ACC_ROWS // PBLK
WBLK = PBLK * QW // 128
SWB = 800
SGRID = FQ // SWB

_mesh = plsc.VectorSubcoreMesh(core_axis_name="c", subcore_axis_name="s")
_sc_params = pltpu.CompilerParams(use_tc_tiling_on_sc=False)


@functools.partial(
    pl.kernel,
    out_type=jax.ShapeDtypeStruct((2, ACC_ROWS, 16), jnp.float32),
    mesh=_mesh,
    scratch_types=[
        pltpu.VMEM((40, EBLK), jnp.int32),
        pltpu.VMEM((EBLK, 16), jnp.float32),
        pltpu.VMEM((EBLK, 16), jnp.float32),
        pltpu.VMEM_SHARED((ACC_ROWS, 16), jnp.float32),
    ],
    compiler_params=_sc_params,
)
def _deg_kernel(row2d, out_hbm, idx_v, ones_v, zero_v, acc_sh):
    cid = lax.axis_index("c")
    sid = lax.axis_index("s")

    def _fill(i, _):
        ones_v[i] = jnp.full((16,), 1.0, jnp.float32)
        zero_v[i] = jnp.zeros((16,), jnp.float32)
        return 0

    lax.fori_loop(0, EBLK, _fill, 0)

    base_row = sid * ROWS_PER_TILE

    def _z(k, _):
        pltpu.sync_copy(zero_v, acc_sh.at[pl.ds(base_row + k * EBLK, EBLK)])
        return 0

    lax.fori_loop(0, 25, _z, 0)
    plsc.subcore_barrier()

    blk0 = cid * (NBLK // 2) + sid * DEG_BLK_PER_TILE

    def _chunk(ci, _):
        pltpu.sync_copy(row2d.at[pl.ds(blk0 + ci * 40, 40)], idx_v)

        def _scat(j, _):
            pltpu.sync_copy(ones_v, acc_sh.at[idx_v.at[j]], add=True)
            return 0

        lax.fori_loop(0, 40, _scat, 0)
        return 0

    lax.fori_loop(0, 5, _chunk, 0)
    plsc.subcore_barrier()

    pltpu.sync_copy(acc_sh.at[pl.ds(base_row, ROWS_PER_TILE)],
                    out_hbm.at[cid, pl.ds(base_row, ROWS_PER_TILE)])


@functools.partial(
    pl.kernel,
    out_type=jax.ShapeDtypeStruct((4, ACC_ROWS, QW), jnp.float32),
    mesh=_mesh,
    scratch_types=(
        [pltpu.VMEM((40, EBLK), jnp.int32)] * 2
        + [pltpu.VMEM((EBLK, QW), jnp.float32)] * 8
        + [pltpu.VMEM_SHARED((ACC_ROWS, QW), jnp.float32)]
        + [pltpu.SemaphoreType.DMA] * 16
    ),
    compiler_params=_sc_params,
)
def _conv_kernel(col2d, row2d, y0, y1, y2, y3, out_hbm,
                 colv, rowv, *rest):
    bufs = rest[:8]
    acc_sh = rest[8]
    gsems = rest[9:17]
    tsems = rest[17:25]
    cid = lax.axis_index("c")
    sid = lax.axis_index("s")
    ys = (y0, y1, y2, y3)
    base_row = sid * ROWS_PER_TILE

    def _wait_scat(i):
        pltpu.make_async_copy(bufs[i], acc_sh.at[rowv.at[0]], tsems[i]).wait()

    blk0 = sid * BLK_PER_TILE

    for p in range(2):
        def _fill(i, _):
            bufs[0][i] = jnp.zeros((16,), jnp.float32)
            return 0

        lax.fori_loop(0, EBLK, _fill, 0)

        def _z(k, _):
            pltpu.sync_copy(bufs[0],
                            acc_sh.at[pl.ds(base_row + k * EBLK, EBLK)])
            return 0

        lax.fori_loop(0, 25, _z, 0)
        plsc.subcore_barrier()

        def _chunk(ci, _):
            @pl.when(ci > 0)
            def _():
                for i in range(8):
                    _wait_scat(i)

            pltpu.sync_copy(col2d.at[pl.ds(blk0 + ci * 40, 40)], colv)
            pltpu.sync_copy(row2d.at[pl.ds(blk0 + ci * 40, 40)], rowv)

            def _group(g, _):
                for par in range(2):
                    @pl.when(g % 2 == par)
                    def _(par=par):
                        b0 = par * 4
                        for i in range(4):
                            jc = g * 4 + i

                            @pl.when(g >= 2)
                            def _(i=i, b0=b0):
                                _wait_scat(b0 + i)

                            @pl.when(cid == 0)
                            def _(jc=jc, k=b0 + i):
                                pltpu.async_copy(ys[p].at[colv.at[jc]],
                                                 bufs[k], gsems[k])

                            @pl.when(cid == 1)
                            def _(jc=jc, k=b0 + i):
                                pltpu.async_copy(ys[2 + p].at[colv.at[jc]],
                                                 bufs[k], gsems[k])
                        for i in range(4):
                            jc = g * 4 + i
                            k = b0 + i
                            pltpu.make_async_copy(
                                ys[p].at[colv.at[jc]], bufs[k],
                                gsems[k]).wait()
                            pltpu.async_copy(
                                bufs[k], acc_sh.at[rowv.at[jc]],
                                tsems[k], add=True)
                return 0

            lax.fori_loop(0, 10, _group, 0)
            return 0

        lax.fori_loop(0, 10, _chunk, 0)
        for i in range(8):
            _wait_scat(i)
        plsc.subcore_barrier()

        pltpu.sync_copy(acc_sh.at[pl.ds(base_row, ROWS_PER_TILE)],
                        out_hbm.at[2 * cid + p, pl.ds(base_row, ROWS_PER_TILE)])


def _n2w(x):
    x3 = jnp.reshape(x, (WBLK, 8, QW))
    return jnp.concatenate([x3[:, s, :] for s in range(8)], axis=1)


def _w2n(w):
    st = jnp.stack([w[:, s * QW:(s + 1) * QW] for s in range(8)], axis=1)
    return jnp.reshape(st, (PBLK, QW))


def _dis_wide(dp0, dp1):
    deg = dp0 + dp1
    return jnp.where(deg > 0, lax.rsqrt(jnp.maximum(deg, 1e-30)), 0.0)


def _mlp_body(feat_ref, pref_ref, w_ref, b_ref, w1_ref, b1_ref, x_ref):
    i = pl.program_id(0)

    def _finish(xb):
        nrm = jnp.maximum(
            jnp.sqrt(jnp.sum(xb * xb, axis=1, keepdims=True)), 1e-12)
        x_ref[...] = xb / nrm

    @pl.when(i < NUM_USER // ROW_BLK)
    def _():
        _finish(pref_ref[...])

    @pl.when(i >= NUM_USER // ROW_BLK)
    def _():
        h = feat_ref[...] @ w_ref[...] + b_ref[0]
        h = jnp.where(h > 0, h, 0.01 * h)
        _finish(h @ w1_ref[...] + b1_ref[0])


def _pack_body(x_ref, dp0_ref, dp1_ref, y0_ref, y1_ref, y2_ref, y3_ref):
    dis = _dis_wide(dp0_ref[...], dp1_ref[...])
    x = x_ref[...]
    for q, yr in enumerate((y0_ref, y1_ref, y2_ref, y3_ref)):
        yr[...] = _n2w(x[:, q * QW:(q + 1) * QW]) * dis


def _scale_body(s0_ref, s1_ref, s2_ref, s3_ref, dp0_ref, dp1_ref,
                h0_ref, h1_ref, h2_ref, h3_ref,
                y0_ref, y1_ref, y2_ref, y3_ref):
    dis = _dis_wide(dp0_ref[...], dp1_ref[...])
    for sr, hr, yr in zip((s0_ref, s1_ref, s2_ref, s3_ref),
                          (h0_ref, h1_ref, h2_ref, h3_ref),
                          (y0_ref, y1_ref, y2_ref, y3_ref)):
        h = sr[...] * dis
        hr[...] = h
        yr[...] = h * dis


def _final_body(x_ref, h0_ref, h1_ref, h2_ref, h3_ref,
                s0_ref, s1_ref, s2_ref, s3_ref, dp0_ref, dp1_ref, out_ref):
    dis = _dis_wide(dp0_ref[...], dp1_ref[...])
    cols = []
    for hr, sr in zip((h0_ref, h1_ref, h2_ref, h3_ref),
                      (s0_ref, s1_ref, s2_ref, s3_ref)):
        cols.append(_w2n(hr[...] + sr[...] * dis))
    out_ref[...] = x_ref[...] + jnp.concatenate(cols, axis=1)


def _wide_spec(plane_of_grid, blk=None):
    b = blk or WBLK
    n = FQ // b
    return pl.BlockSpec((b, 128), lambda i, p=plane_of_grid, n=n: (p * n + i, 0))


def _q_spec():
    return pl.BlockSpec((WBLK, 128), lambda i: (i, 0))


def _x_spec():
    return pl.BlockSpec((PBLK, DIM_LATENT), lambda i: (i, 0))


def _q_outs():
    return [jax.ShapeDtypeStruct((FQ, 128), jnp.float32) for _ in range(4)]


def kernel(edge_index_drop, edge_index, features, preference,
           W_mlp, b_mlp, W_mlp1, b_mlp1):
    del edge_index_drop
    row = edge_index[0].astype(jnp.int32)
    col = edge_index[1].astype(jnp.int32)
    pad = PAD_E - N_EDGES
    row2d = jnp.concatenate(
        [row, jnp.full((pad,), DUMMY_ROW, jnp.int32)]).reshape(NBLK, EBLK)
    col2d = jnp.concatenate(
        [col, jnp.zeros((pad,), jnp.int32)]).reshape(NBLK, EBLK)

    degp = _deg_kernel(row2d)
    degp_flat = jnp.reshape(degp, (2 * FQ, 128))

    x = pl.pallas_call(
        _mlp_body,
        grid=(GRID,),
        in_specs=[
            pl.BlockSpec((ROW_BLK, DIM_FEAT),
                         lambda i: (jnp.clip(i - NUM_USER // ROW_BLK, 0,
                                             NUM_ITEM // ROW_BLK - 1), 0)),
            pl.BlockSpec((ROW_BLK, DIM_LATENT),
                         lambda i: (jnp.minimum(i, NUM_USER // ROW_BLK - 1), 0)),
            pl.BlockSpec((DIM_FEAT, 4 * DIM_LATENT), lambda i: (0, 0)),
            pl.BlockSpec((1, 4 * DIM_LATENT), lambda i: (0, 0)),
            pl.BlockSpec((4 * DIM_LATENT, DIM_LATENT), lambda i: (0, 0)),
            pl.BlockSpec((1, DIM_LATENT), lambda i: (0, 0)),
        ],
        out_specs=pl.BlockSpec((ROW_BLK, DIM_LATENT), lambda i: (i, 0)),
        out_shape=jax.ShapeDtypeStruct((N_NODES, DIM_LATENT), jnp.float32),
    )(features, preference, W_mlp, b_mlp.reshape(1, -1),
      W_mlp1, b_mlp1.reshape(1, -1))

    ya = pl.pallas_call(
        _pack_body,
        grid=(PGRID,),
        in_specs=[_x_spec(), _wide_spec(0), _wide_spec(1)],
        out_specs=[_q_spec()] * 4,
        out_shape=_q_outs(),
    )(x, degp_flat, degp_flat)

    s1 = _conv_kernel(col2d, row2d,
                      *(jnp.reshape(y, (ACC_ROWS, QW)) for y in ya))
    s1_flat = jnp.reshape(s1, (4 * FQ, 128))

    h_and_y2 = pl.pallas_call(
        _scale_body,
        grid=(SGRID,),
        in_specs=[_wide_spec(0, SWB), _wide_spec(1, SWB), _wide_spec(2, SWB),
                  _wide_spec(3, SWB), _wide_spec(0, SWB), _wide_spec(1, SWB)],
        out_specs=[pl.BlockSpec((SWB, 128), lambda i: (i, 0))] * 8,
        out_shape=_q_outs() + _q_outs(),
    )(s1_flat, s1_flat, s1_flat, s1_flat, degp_flat, degp_flat)
    hq, yb = h_and_y2[:4], h_and_y2[4:]

    s2 = _conv_kernel(col2d, row2d,
                      *(jnp.reshape(y, (ACC_ROWS, QW)) for y in yb))
    s2_flat = jnp.reshape(s2, (4 * FQ, 128))

    x_hat = pl.pallas_call(
        _final_body,
        grid=(PGRID,),
        in_specs=[_x_spec()] + [_q_spec()] * 4
        + [_wide_spec(0), _wide_spec(1), _wide_spec(2), _wide_spec(3),
           _wide_spec(0), _wide_spec(1)],
        out_specs=_x_spec(),
        out_shape=jax.ShapeDtypeStruct((N_NODES, DIM_LATENT), jnp.float32),
    )(x, *hq, s2_flat, s2_flat, s2_flat, s2_flat, degp_flat, degp_flat)

    return (x_hat, preference)


# speedup vs baseline: 16.2825x; 1.1260x over previous
import functools

import jax
import jax.numpy as jnp
from jax import lax
from jax.experimental import pallas as pl
from jax.experimental.pallas import tpu as pltpu
from jax.experimental.pallas import tpu_sc as plsc

NUM_USER = 20000
NUM_ITEM = 30000
N_NODES = NUM_USER + NUM_ITEM
N_EDGES = 800000
DIM_FEAT = 128
DIM_LATENT = 64
QW = 16

EBLK = 128
NBLK = 6400
PAD_E = NBLK * EBLK
BLK_PER_TILE = NBLK // 16
DEG_BLK_PER_TILE = NBLK // 32
ACC_ROWS = 51200
ROWS_PER_TILE = ACC_ROWS // 16
DUMMY_ROW = N_NODES
FQ = ACC_ROWS * QW // 128
ROW_BLK = 2000
GRID = N_NODES // ROW_BLK
PBLK = 1600
PGRID = ACC_ROWS // PBLK
WBLK = PBLK * QW // 128
SWB = 800
SGRID = FQ // SWB

_mesh = plsc.VectorSubcoreMesh(core_axis_name="c", subcore_axis_name="s")
_sc_params = pltpu.CompilerParams(use_tc_tiling_on_sc=False)


@functools.partial(
    pl.kernel,
    out_type=jax.ShapeDtypeStruct((2, ACC_ROWS, 16), jnp.float32),
    mesh=_mesh,
    scratch_types=[
        pltpu.VMEM((40, EBLK), jnp.int32),
        pltpu.VMEM((EBLK, 16), jnp.float32),
        pltpu.VMEM((EBLK, 16), jnp.float32),
        pltpu.VMEM_SHARED((ACC_ROWS, 16), jnp.float32),
    ],
    compiler_params=_sc_params,
)
def _deg_kernel(row2d, out_hbm, idx_v, ones_v, zero_v, acc_sh):
    cid = lax.axis_index("c")
    sid = lax.axis_index("s")

    def _fill(i, _):
        ones_v[i] = jnp.full((16,), 1.0, jnp.float32)
        zero_v[i] = jnp.zeros((16,), jnp.float32)
        return 0

    lax.fori_loop(0, EBLK, _fill, 0)

    base_row = sid * ROWS_PER_TILE

    def _z(k, _):
        pltpu.sync_copy(zero_v, acc_sh.at[pl.ds(base_row + k * EBLK, EBLK)])
        return 0

    lax.fori_loop(0, 25, _z, 0)
    plsc.subcore_barrier()

    blk0 = cid * (NBLK // 2) + sid * DEG_BLK_PER_TILE

    def _chunk(ci, _):
        pltpu.sync_copy(row2d.at[pl.ds(blk0 + ci * 40, 40)], idx_v)

        def _scat(j, _):
            pltpu.sync_copy(ones_v, acc_sh.at[idx_v.at[j]], add=True)
            return 0

        lax.fori_loop(0, 40, _scat, 0)
        return 0

    lax.fori_loop(0, 5, _chunk, 0)
    plsc.subcore_barrier()

    pltpu.sync_copy(acc_sh.at[pl.ds(base_row, ROWS_PER_TILE)],
                    out_hbm.at[cid, pl.ds(base_row, ROWS_PER_TILE)])


@functools.partial(
    pl.kernel,
    out_type=jax.ShapeDtypeStruct((4, ACC_ROWS, QW), jnp.float32),
    mesh=_mesh,
    scratch_types=(
        [pltpu.VMEM((40, EBLK), jnp.int32)] * 2
        + [pltpu.VMEM((EBLK, QW), jnp.float32)] * 8
        + [pltpu.VMEM_SHARED((ACC_ROWS, QW), jnp.float32)]
        + [pltpu.SemaphoreType.DMA] * 16
    ),
    compiler_params=_sc_params,
)
def _conv_kernel(col2d, row2d, y0, y1, y2, y3, out_hbm,
                 colv, rowv, *rest):
    bufs = rest[:8]
    acc_sh = rest[8]
    gsems = rest[9:17]
    tsems = rest[17:25]
    cid = lax.axis_index("c")
    sid = lax.axis_index("s")
    ys = (y0, y1, y2, y3)
    base_row = sid * ROWS_PER_TILE

    def _wait_scat(i):
        pltpu.make_async_copy(bufs[i], acc_sh.at[rowv.at[0]], tsems[i]).wait()

    blk0 = sid * BLK_PER_TILE

    for p in range(2):
        def _fill(i, _):
            bufs[0][i] = jnp.zeros((16,), jnp.float32)
            return 0

        lax.fori_loop(0, EBLK, _fill, 0)

        def _z(k, _):
            pltpu.sync_copy(bufs[0],
                            acc_sh.at[pl.ds(base_row + k * EBLK, EBLK)])
            return 0

        lax.fori_loop(0, 25, _z, 0)
        plsc.subcore_barrier()

        def _chunk(ci, _):
            @pl.when(ci > 0)
            def _():
                for i in range(8):
                    _wait_scat(i)

            pltpu.sync_copy(col2d.at[pl.ds(blk0 + ci * 40, 40)], colv)
            pltpu.sync_copy(row2d.at[pl.ds(blk0 + ci * 40, 40)], rowv)

            def _group(g, _):
                for par in range(2):
                    @pl.when(g % 2 == par)
                    def _(par=par):
                        b0 = par * 4
                        for i in range(4):
                            jc = g * 4 + i

                            @pl.when(g >= 2)
                            def _(i=i, b0=b0):
                                _wait_scat(b0 + i)

                            @pl.when(cid == 0)
                            def _(jc=jc, k=b0 + i):
                                pltpu.async_copy(ys[p].at[colv.at[jc]],
                                                 bufs[k], gsems[k])

                            @pl.when(cid == 1)
                            def _(jc=jc, k=b0 + i):
                                pltpu.async_copy(ys[2 + p].at[colv.at[jc]],
                                                 bufs[k], gsems[k])
                        for i in range(4):
                            jc = g * 4 + i
                            k = b0 + i
                            pltpu.make_async_copy(
                                ys[p].at[colv.at[jc]], bufs[k],
                                gsems[k]).wait()
                            pltpu.async_copy(
                                bufs[k], acc_sh.at[rowv.at[jc]],
                                tsems[k], add=True)
                return 0

            lax.fori_loop(0, 10, _group, 0)
            return 0

        lax.fori_loop(0, 10, _chunk, 0)
        for i in range(8):
            _wait_scat(i)
        plsc.subcore_barrier()

        pltpu.sync_copy(acc_sh.at[pl.ds(base_row, ROWS_PER_TILE)],
                        out_hbm.at[2 * cid + p, pl.ds(base_row, ROWS_PER_TILE)])


def _n2w(x):
    x3 = jnp.reshape(x, (WBLK, 8, QW))
    return jnp.concatenate([x3[:, s, :] for s in range(8)], axis=1)


def _w2n(w):
    st = jnp.stack([w[:, s * QW:(s + 1) * QW] for s in range(8)], axis=1)
    return jnp.reshape(st, (PBLK, QW))


def _dis_wide(dp0, dp1):
    deg = dp0 + dp1
    return jnp.where(deg > 0, lax.rsqrt(jnp.maximum(deg, 1e-30)), 0.0)


def _mlp_body(feat_ref, pref_ref, w_ref, b_ref, w1_ref, b1_ref, x_ref):
    i = pl.program_id(0)

    def _finish(xb):
        nrm = jnp.maximum(
            jnp.sqrt(jnp.sum(xb * xb, axis=1, keepdims=True)), 1e-12)
        x_ref[...] = xb / nrm

    @pl.when(i < NUM_USER // ROW_BLK)
    def _():
        _finish(pref_ref[...])

    @pl.when(i >= NUM_USER // ROW_BLK)
    def _():
        h = feat_ref[...] @ w_ref[...] + b_ref[0]
        h = jnp.where(h > 0, h, 0.01 * h)
        _finish(h @ w1_ref[...] + b1_ref[0])


def _pack_body(x_ref, dp0_ref, dp1_ref, x0_ref, x1_ref, x2_ref, x3_ref,
               y0_ref, y1_ref, y2_ref, y3_ref):
    dis = _dis_wide(dp0_ref[...], dp1_ref[...])
    x = x_ref[...]
    for q, (xr, yr) in enumerate(zip((x0_ref, x1_ref, x2_ref, x3_ref),
                                     (y0_ref, y1_ref, y2_ref, y3_ref))):
        xq = _n2w(x[:, q * QW:(q + 1) * QW])
        xr[...] = xq
        yr[...] = xq * dis


def _scale_body(s0_ref, s1_ref, s2_ref, s3_ref, dp0_ref, dp1_ref,
                h0_ref, h1_ref, h2_ref, h3_ref,
                y0_ref, y1_ref, y2_ref, y3_ref):
    dis = _dis_wide(dp0_ref[...], dp1_ref[...])
    for sr, hr, yr in zip((s0_ref, s1_ref, s2_ref, s3_ref),
                          (h0_ref, h1_ref, h2_ref, h3_ref),
                          (y0_ref, y1_ref, y2_ref, y3_ref)):
        h = sr[...] * dis
        hr[...] = h
        yr[...] = h * dis


def _final_body(x0_ref, x1_ref, x2_ref, x3_ref, h0_ref, h1_ref, h2_ref,
                h3_ref, s0_ref, s1_ref, s2_ref, s3_ref, dp0_ref, dp1_ref,
                o0_ref, o1_ref, o2_ref, o3_ref):
    dis = _dis_wide(dp0_ref[...], dp1_ref[...])
    for xr, hr, sr, orf in zip((x0_ref, x1_ref, x2_ref, x3_ref),
                               (h0_ref, h1_ref, h2_ref, h3_ref),
                               (s0_ref, s1_ref, s2_ref, s3_ref),
                               (o0_ref, o1_ref, o2_ref, o3_ref)):
        orf[...] = xr[...] + hr[...] + sr[...] * dis


def _wide_spec(plane_of_grid, blk=None):
    b = blk or WBLK
    n = FQ // b
    return pl.BlockSpec((b, 128), lambda i, p=plane_of_grid, n=n: (p * n + i, 0))


def _q_spec():
    return pl.BlockSpec((WBLK, 128), lambda i: (i, 0))


def _x_spec():
    return pl.BlockSpec((PBLK, DIM_LATENT), lambda i: (i, 0))


def _q_outs():
    return [jax.ShapeDtypeStruct((FQ, 128), jnp.float32) for _ in range(4)]


def kernel(edge_index_drop, edge_index, features, preference,
           W_mlp, b_mlp, W_mlp1, b_mlp1):
    del edge_index_drop
    row = edge_index[0].astype(jnp.int32)
    col = edge_index[1].astype(jnp.int32)
    pad = PAD_E - N_EDGES
    row2d = jnp.concatenate(
        [row, jnp.full((pad,), DUMMY_ROW, jnp.int32)]).reshape(NBLK, EBLK)
    col2d = jnp.concatenate(
        [col, jnp.zeros((pad,), jnp.int32)]).reshape(NBLK, EBLK)

    degp = _deg_kernel(row2d)
    degp_flat = jnp.reshape(degp, (2 * FQ, 128))

    x = pl.pallas_call(
        _mlp_body,
        grid=(GRID,),
        in_specs=[
            pl.BlockSpec((ROW_BLK, DIM_FEAT),
                         lambda i: (jnp.clip(i - NUM_USER // ROW_BLK, 0,
                                             NUM_ITEM // ROW_BLK - 1), 0)),
            pl.BlockSpec((ROW_BLK, DIM_LATENT),
                         lambda i: (jnp.minimum(i, NUM_USER // ROW_BLK - 1), 0)),
            pl.BlockSpec((DIM_FEAT, 4 * DIM_LATENT), lambda i: (0, 0)),
            pl.BlockSpec((1, 4 * DIM_LATENT), lambda i: (0, 0)),
            pl.BlockSpec((4 * DIM_LATENT, DIM_LATENT), lambda i: (0, 0)),
            pl.BlockSpec((1, DIM_LATENT), lambda i: (0, 0)),
        ],
        out_specs=pl.BlockSpec((ROW_BLK, DIM_LATENT), lambda i: (i, 0)),
        out_shape=jax.ShapeDtypeStruct((N_NODES, DIM_LATENT), jnp.float32),
    )(features, preference, W_mlp, b_mlp.reshape(1, -1),
      W_mlp1, b_mlp1.reshape(1, -1))

    xq_ya = pl.pallas_call(
        _pack_body,
        grid=(PGRID,),
        in_specs=[_x_spec(), _wide_spec(0), _wide_spec(1)],
        out_specs=[_q_spec()] * 8,
        out_shape=_q_outs() + _q_outs(),
    )(x, degp_flat, degp_flat)
    xq, ya = xq_ya[:4], xq_ya[4:]

    s1 = _conv_kernel(col2d, row2d,
                      *(jnp.reshape(y, (ACC_ROWS, QW)) for y in ya))
    s1_flat = jnp.reshape(s1, (4 * FQ, 128))

    h_and_y2 = pl.pallas_call(
        _scale_body,
        grid=(SGRID,),
        in_specs=[_wide_spec(0, SWB), _wide_spec(1, SWB), _wide_spec(2, SWB),
                  _wide_spec(3, SWB), _wide_spec(0, SWB), _wide_spec(1, SWB)],
        out_specs=[pl.BlockSpec((SWB, 128), lambda i: (i, 0))] * 8,
        out_shape=_q_outs() + _q_outs(),
    )(s1_flat, s1_flat, s1_flat, s1_flat, degp_flat, degp_flat)
    hq, yb = h_and_y2[:4], h_and_y2[4:]

    s2 = _conv_kernel(col2d, row2d,
                      *(jnp.reshape(y, (ACC_ROWS, QW)) for y in yb))
    s2_flat = jnp.reshape(s2, (4 * FQ, 128))

    sspec = pl.BlockSpec((SWB, 128), lambda i: (i, 0))
    oq = pl.pallas_call(
        _final_body,
        grid=(SGRID,),
        in_specs=[sspec] * 8
        + [_wide_spec(0, SWB), _wide_spec(1, SWB), _wide_spec(2, SWB),
           _wide_spec(3, SWB), _wide_spec(0, SWB), _wide_spec(1, SWB)],
        out_specs=[sspec] * 4,
        out_shape=_q_outs(),
    )(*xq, *hq, s2_flat, s2_flat, s2_flat, s2_flat, degp_flat, degp_flat)

    x_hat = jnp.concatenate(
        [jnp.reshape(q, (ACC_ROWS, QW)) for q in oq], axis=1)[:N_NODES]
    return (x_hat, preference)
